# baseline (speedup 1.0000x reference)
import jax
import jax.numpy as jnp
from jax.experimental import pallas as pl

_STRIDE = 32
_TOPK = 4


def _body(hs_ref, m_ref, q_ref, out_ref, idx_ref):
    M = hs_ref.shape[1]
    x = hs_ref[0]
    q = q_ref[0]
    m = m_ref[0][:, 0:1]

    dots = jnp.sum(x * q, axis=1, keepdims=True)
    n2 = jnp.sum(x * x, axis=1, keepdims=True)
    qn = jnp.sqrt(jnp.sum(q * q, keepdims=True))
    denom = jnp.maximum(jnp.sqrt(n2), 1e-12) * jnp.maximum(qn, 1e-12)
    neg = jnp.float32(-jnp.inf)
    scores = jnp.where(m > 0, dots / denom, neg)

    sidx = jax.lax.broadcasted_iota(jnp.int32, (M, 1), 0)

    cur = scores
    top_i = []
    top_v = []
    for _ in range(_TOPK):
        bv = jnp.max(cur, keepdims=True)
        bi = jnp.min(jnp.where(cur == bv, sidx, M), keepdims=True)
        top_v.append(bv)
        top_i.append(bi)
        cur = jnp.where(sidx == bi, neg, cur)

    sel = (sidx == top_i[0]) | (sidx == top_i[1]) | (sidx == top_i[2]) | (
        sidx == top_i[3])
    e = jnp.where(sel, jnp.exp(scores - top_v[0]), 0.0)
    wv = e / jnp.sum(e, keepdims=True)

    out_ref[0, 0, :] = jnp.sum(x * wv, axis=0)

    lane = jax.lax.broadcasted_iota(jnp.int32, (8, 128), 1)
    row = jnp.where(lane == 0, jnp.broadcast_to(top_i[0], (8, 128)),
          jnp.where(lane == 1, jnp.broadcast_to(top_i[1], (8, 128)),
          jnp.where(lane == 2, jnp.broadcast_to(top_i[2], (8, 128)),
          jnp.where(lane == 3, jnp.broadcast_to(top_i[3], (8, 128)), 0))))
    idx_ref[0] = row


def kernel(hidden_states, attention_mask, query):
    B, S, D = hidden_states.shape
    M = S // _STRIDE
    hs3 = hidden_states.reshape(B, M, _STRIDE * D)
    mask3 = attention_mask.reshape(B, M, _STRIDE).astype(jnp.float32)
    q3 = query.reshape(B, 1, D)

    retrieved, idx_pad = pl.pallas_call(
        _body,
        grid=(B,),
        in_specs=[
            pl.BlockSpec((1, M, D), lambda b: (b, 0, 0)),
            pl.BlockSpec((1, M, _STRIDE), lambda b: (b, 0, 0)),
            pl.BlockSpec((1, 1, D), lambda b: (b, 0, 0)),
        ],
        out_specs=[
            pl.BlockSpec((1, 1, D), lambda b: (b, 0, 0)),
            pl.BlockSpec((1, 8, 128), lambda b: (b, 0, 0)),
        ],
        out_shape=[
            jax.ShapeDtypeStruct((B, 1, D), jnp.float32),
            jax.ShapeDtypeStruct((B, 8, 128), jnp.int32),
        ],
    )(hs3, mask3, q3)

    return retrieved.reshape(B, D), idx_pad[:, 0, :_TOPK]


# speedup vs baseline: 8.4766x; 8.4766x over previous
import dataclasses
import functools

import jax
import jax.numpy as jnp
from jax import lax
from jax.experimental import pallas as pl
from jax.experimental.pallas import tpu as pltpu
from jax.experimental.pallas import tpu_sc as plsc

_STRIDE = 32
_TOPK = 4
_L = 16
_NTILE = 32


def _sc_gather(hs2, n_rows, D):
    per_tile = n_rows // _NTILE
    n_chunk = per_tile // _STRIDE

    mesh = plsc.VectorSubcoreMesh(core_axis_name="c", subcore_axis_name="s")
    cp = pltpu.CompilerParams()
    if "needs_layout_passes" in pltpu.CompilerParams.__dataclass_fields__:
        cp = dataclasses.replace(cp, needs_layout_passes=False)

    @functools.partial(
        pl.kernel,
        mesh=mesh,
        compiler_params=cp,
        out_type=jax.ShapeDtypeStruct((n_rows, D), jnp.float32),
        scratch_types=[
            pltpu.VMEM((2 * _L,), jnp.int32),
            pltpu.VMEM((_STRIDE, D), jnp.float32),
        ],
    )
    def gather_kernel(hs_hbm, out_hbm, idxc_v, rowbuf):
        c = lax.axis_index("c")
        s = lax.axis_index("s")
        w = c * 16 + s
        iota = lax.broadcasted_iota(jnp.int32, (_L,), 0)

        @pl.loop(0, n_chunk)
        def _chunk(ch):
            f0 = w * per_tile + ch * _STRIDE
            idxc_v[pl.ds(0, _L)] = (f0 + iota) * _STRIDE
            idxc_v[pl.ds(_L, _L)] = (f0 + _L + iota) * _STRIDE
            pltpu.sync_copy(hs_hbm.at[idxc_v], rowbuf)
            pltpu.sync_copy(rowbuf, out_hbm.at[pl.ds(f0, _STRIDE)])

    return gather_kernel(hs2)


def _tc_body(x_ref, m_ref, q_ref, out_ref, idx_ref):
    M = x_ref.shape[1]
    x = x_ref[0]
    q = q_ref[0]
    m = m_ref[0][:, 0:1]

    qn = jnp.sqrt(jnp.sum(q * q, keepdims=True))
    nq = q / jnp.maximum(qn, 1e-12)
    n2 = jnp.sum(x * x, axis=1, keepdims=True)
    nm = x / jnp.maximum(jnp.sqrt(n2), 1e-12)
    nq8 = jnp.broadcast_to(nq.astype(jnp.bfloat16), (8, nq.shape[1]))
    scores = jax.lax.dot_general(
        nm.astype(jnp.bfloat16), nq8,
        (((1,), (1,)), ((), ())),
        preferred_element_type=jnp.float32)[:, 0:1]
    neg = jnp.float32(-jnp.inf)
    scores = jnp.where(m > 0, scores, neg)

    sidx = jax.lax.broadcasted_iota(jnp.int32, (M, 1), 0)

    cur = scores
    top_i = []
    top_v = []
    for _ in range(_TOPK):
        bv = jnp.max(cur, keepdims=True)
        bi = jnp.min(jnp.where(cur == bv, sidx, M), keepdims=True)
        top_v.append(bv)
        top_i.append(bi)
        cur = jnp.where(sidx == bi, neg, cur)

    sel = (sidx == top_i[0]) | (sidx == top_i[1]) | (sidx == top_i[2]) | (
        sidx == top_i[3])
    e = jnp.where(sel, jnp.exp(scores - top_v[0]), 0.0)
    wv = e / jnp.sum(e, keepdims=True)

    out_ref[0, 0, :] = jnp.sum(x * wv, axis=0)

    lane = jax.lax.broadcasted_iota(jnp.int32, (8, 128), 1)
    row = jnp.where(lane == 0, jnp.broadcast_to(top_i[0], (8, 128)),
          jnp.where(lane == 1, jnp.broadcast_to(top_i[1], (8, 128)),
          jnp.where(lane == 2, jnp.broadcast_to(top_i[2], (8, 128)),
          jnp.where(lane == 3, jnp.broadcast_to(top_i[3], (8, 128)), 0))))
    idx_ref[0] = row


def kernel(hidden_states, attention_mask, query):
    B, S, D = hidden_states.shape
    M = S // _STRIDE

    hs2 = hidden_states.reshape(B * S, D)
    compact = _sc_gather(hs2, B * M, D).reshape(B, M, D)

    mask3 = attention_mask.reshape(B, M, _STRIDE).astype(jnp.float32)
    q3 = query.reshape(B, 1, D)

    retrieved, idx_pad = pl.pallas_call(
        _tc_body,
        grid=(B,),
        in_specs=[
            pl.BlockSpec((1, M, D), lambda b: (b, 0, 0)),
            pl.BlockSpec((1, M, _STRIDE), lambda b: (b, 0, 0)),
            pl.BlockSpec((1, 1, D), lambda b: (b, 0, 0)),
        ],
        out_specs=[
            pl.BlockSpec((1, 1, D), lambda b: (b, 0, 0)),
            pl.BlockSpec((1, 8, 128), lambda b: (b, 0, 0)),
        ],
        out_shape=[
            jax.ShapeDtypeStruct((B, 1, D), jnp.float32),
            jax.ShapeDtypeStruct((B, 8, 128), jnp.int32),
        ],
    )(compact, mask3, q3)

    return retrieved.reshape(B, D), idx_pad[:, 0, :_TOPK]


# speedup vs baseline: 12.5829x; 1.4844x over previous
import dataclasses
import functools

import jax
import jax.numpy as jnp
from jax import lax
from jax.experimental import pallas as pl
from jax.experimental.pallas import tpu as pltpu
from jax.experimental.pallas import tpu_sc as plsc

_STRIDE = 32
_TOPK = 4
_L = 16
_NTILE = 32


def _sc_gather(hs2, n_rows, D):
    per_tile = n_rows // _NTILE
    n_chunk = per_tile // _STRIDE

    mesh = plsc.VectorSubcoreMesh(core_axis_name="c", subcore_axis_name="s")
    cp = pltpu.CompilerParams()
    if "needs_layout_passes" in pltpu.CompilerParams.__dataclass_fields__:
        cp = dataclasses.replace(cp, needs_layout_passes=False)

    @functools.partial(
        pl.kernel,
        mesh=mesh,
        compiler_params=cp,
        out_type=jax.ShapeDtypeStruct((n_rows, D), jnp.float32),
        scratch_types=[
            pltpu.VMEM((2 * _L,), jnp.int32),
            pltpu.VMEM((_STRIDE, D), jnp.float32),
        ],
    )
    def gather_kernel(hs_hbm, out_hbm, idxc_v, rowbuf):
        c = lax.axis_index("c")
        s = lax.axis_index("s")
        w = c * 16 + s
        iota = lax.broadcasted_iota(jnp.int32, (_L,), 0)

        @pl.loop(0, n_chunk)
        def _chunk(ch):
            f0 = w * per_tile + ch * _STRIDE
            idxc_v[pl.ds(0, _L)] = (f0 + iota) * _STRIDE
            idxc_v[pl.ds(_L, _L)] = (f0 + _L + iota) * _STRIDE
            pltpu.sync_copy(hs_hbm.at[idxc_v], rowbuf)
            pltpu.sync_copy(rowbuf, out_hbm.at[pl.ds(f0, _STRIDE)])

    return gather_kernel(hs2)


def _tc_body(x_ref, m_ref, q_ref, out_ref, idx_ref):
    BB, M, D = x_ref.shape
    x_all = x_ref[...]
    q_all = q_ref[...]
    mrow = m_ref[:, 0, :]

    qn = jnp.sqrt(jnp.sum(q_all * q_all, axis=2, keepdims=True))
    nq = q_all / jnp.maximum(qn, 1e-12)
    n2 = jnp.sum(x_all * x_all, axis=2, keepdims=True)
    nm = x_all / jnp.maximum(jnp.sqrt(n2), 1e-12)
    nm_bf = nm.astype(jnp.bfloat16)
    nq_bf = nq.astype(jnp.bfloat16)
    rows = []
    for bb in range(BB):
        nq8 = jnp.broadcast_to(nq_bf[bb], (8, D))
        s8 = jax.lax.dot_general(
            nq8, nm_bf[bb], (((1,), (1,)), ((), ())),
            preferred_element_type=jnp.float32)
        rows.append(s8[0:1, :])
    neg = jnp.float32(-jnp.inf)
    s = jnp.where(mrow > 0, jnp.concatenate(rows, axis=0), neg)

    lidx = jax.lax.broadcasted_iota(jnp.int32, (BB, M), 1)

    cur = s
    top_i = []
    top_v = []
    for _ in range(_TOPK):
        bv = jnp.max(cur, axis=1, keepdims=True)
        bi = jnp.min(jnp.where(cur == bv, lidx, M), axis=1, keepdims=True)
        top_v.append(bv)
        top_i.append(bi)
        cur = jnp.where(lidx == bi, neg, cur)

    sel = (lidx == top_i[0]) | (lidx == top_i[1]) | (lidx == top_i[2]) | (
        lidx == top_i[3])
    e = jnp.where(sel, jnp.exp(s - top_v[0]), 0.0)
    esum = jnp.sum(e, axis=1, keepdims=True)
    w = [jnp.sum(jnp.where(lidx == top_i[t], e, 0.0), axis=1, keepdims=True)
         / esum for t in range(_TOPK)]

    sidxc = jax.lax.broadcasted_iota(jnp.int32, (M, 1), 0)
    lane = jax.lax.broadcasted_iota(jnp.int32, (8, 128), 1)
    for bb in range(BB):
        wcol = jnp.where(sidxc == top_i[0][bb:bb + 1], w[0][bb:bb + 1], 0.0)
        for t in range(1, _TOPK):
            wcol = wcol + jnp.where(sidxc == top_i[t][bb:bb + 1],
                                    w[t][bb:bb + 1], 0.0)
        out_ref[bb, 0, :] = jnp.sum(x_all[bb] * wcol, axis=0)

        row = jnp.where(
            lane == 0, jnp.broadcast_to(top_i[0][bb:bb + 1], (8, 128)),
            jnp.where(
                lane == 1, jnp.broadcast_to(top_i[1][bb:bb + 1], (8, 128)),
                jnp.where(
                    lane == 2, jnp.broadcast_to(top_i[2][bb:bb + 1], (8, 128)),
                    jnp.where(lane == 3,
                              jnp.broadcast_to(top_i[3][bb:bb + 1], (8, 128)),
                              0))))
        idx_ref[bb] = row


def kernel(hidden_states, attention_mask, query):
    B, S, D = hidden_states.shape
    M = S // _STRIDE

    hs2 = hidden_states.reshape(B * S, D)
    compact = _sc_gather(hs2, B * M, D).reshape(B, M, D)

    mask3 = attention_mask.reshape(B, M, _STRIDE).astype(
        jnp.float32).transpose(0, 2, 1)
    q3 = query.reshape(B, 1, D)

    BB = 4
    retrieved, idx_pad = pl.pallas_call(
        _tc_body,
        grid=(B // BB,),
        in_specs=[
            pl.BlockSpec((BB, M, D), lambda b: (b, 0, 0)),
            pl.BlockSpec((BB, _STRIDE, M), lambda b: (b, 0, 0)),
            pl.BlockSpec((BB, 1, D), lambda b: (b, 0, 0)),
        ],
        out_specs=[
            pl.BlockSpec((BB, 1, D), lambda b: (b, 0, 0)),
            pl.BlockSpec((BB, 8, 128), lambda b: (b, 0, 0)),
        ],
        out_shape=[
            jax.ShapeDtypeStruct((B, 1, D), jnp.float32),
            jax.ShapeDtypeStruct((B, 8, 128), jnp.int32),
        ],
    )(compact, mask3, q3)

    return retrieved.reshape(B, D), idx_pad[:, 0, :_TOPK]


# speedup vs baseline: 12.8441x; 1.0208x over previous
import dataclasses
import functools

import jax
import jax.numpy as jnp
from jax import lax
from jax.experimental import pallas as pl
from jax.experimental.pallas import tpu as pltpu
from jax.experimental.pallas import tpu_sc as plsc

_STRIDE = 32
_TOPK = 4
_L = 16
_NTILE = 32


def _sc_gather(hs2, n_rows, D):
    per_tile = n_rows // _NTILE
    n_chunk = per_tile // _STRIDE

    mesh = plsc.VectorSubcoreMesh(core_axis_name="c", subcore_axis_name="s")
    cp = pltpu.CompilerParams()
    if "needs_layout_passes" in pltpu.CompilerParams.__dataclass_fields__:
        cp = dataclasses.replace(cp, needs_layout_passes=False)

    @functools.partial(
        pl.kernel,
        mesh=mesh,
        compiler_params=cp,
        out_type=jax.ShapeDtypeStruct((n_rows, D), jnp.float32),
        scratch_types=[
            pltpu.VMEM((2 * _L,), jnp.int32),
            pltpu.VMEM((2 * _L,), jnp.int32),
            pltpu.VMEM((_STRIDE, D), jnp.float32),
            pltpu.VMEM((_STRIDE, D), jnp.float32),
            pltpu.SemaphoreType.DMA,
            pltpu.SemaphoreType.DMA,
            pltpu.SemaphoreType.DMA,
            pltpu.SemaphoreType.DMA,
        ],
    )
    def gather_kernel(hs_hbm, out_hbm, idx_a, idx_b, buf_a, buf_b,
                      sg0, sg1, sw0, sw1):
        c = lax.axis_index("c")
        s = lax.axis_index("s")
        w = c * 16 + s
        iota = lax.broadcasted_iota(jnp.int32, (_L,), 0)
        idx = [idx_a, idx_b]
        buf = [buf_a, buf_b]
        sg = [sg0, sg1]
        sw = [sw0, sw1]

        def mkidx(ch, ib):
            f0 = w * per_tile + ch * _STRIDE
            ib[pl.ds(0, _L)] = (f0 + iota) * _STRIDE
            ib[pl.ds(_L, _L)] = (f0 + _L + iota) * _STRIDE

        gathers = [None] * n_chunk
        writes = [None] * n_chunk
        mkidx(0, idx[0])
        gathers[0] = pltpu.async_copy(hs_hbm.at[idx[0]], buf[0], sg[0])
        for ch in range(n_chunk):
            p = ch % 2
            if ch + 1 < n_chunk:
                mkidx(ch + 1, idx[1 - p])
                if ch >= 1:
                    writes[ch - 1].wait()
                gathers[ch + 1] = pltpu.async_copy(
                    hs_hbm.at[idx[1 - p]], buf[1 - p], sg[1 - p])
            gathers[ch].wait()
            f0 = w * per_tile + ch * _STRIDE
            writes[ch] = pltpu.async_copy(
                buf[p], out_hbm.at[pl.ds(f0, _STRIDE)], sw[p])
        writes[n_chunk - 2].wait()
        writes[n_chunk - 1].wait()

    return gather_kernel(hs2)


def _tc_body(x_ref, m_ref, q_ref, out_ref, idx_ref):
    BB, M, D = x_ref.shape
    x_all = x_ref[...]
    q_all = q_ref[...]
    mrow = m_ref[:, 0, :]

    qn = jnp.sqrt(jnp.sum(q_all * q_all, axis=2, keepdims=True))
    nq = q_all / jnp.maximum(qn, 1e-12)
    n2 = jnp.sum(x_all * x_all, axis=2, keepdims=True)
    nm = x_all / jnp.maximum(jnp.sqrt(n2), 1e-12)
    nm_bf = nm.astype(jnp.bfloat16)
    nq_bf = nq.astype(jnp.bfloat16)
    rows = []
    for bb in range(BB):
        nq8 = jnp.broadcast_to(nq_bf[bb], (8, D))
        s8 = jax.lax.dot_general(
            nq8, nm_bf[bb], (((1,), (1,)), ((), ())),
            preferred_element_type=jnp.float32)
        rows.append(s8[0:1, :])
    neg = jnp.float32(-jnp.inf)
    s = jnp.where(mrow > 0, jnp.concatenate(rows, axis=0), neg)

    lidx = jax.lax.broadcasted_iota(jnp.int32, (BB, M), 1)

    cur = s
    top_i = []
    top_v = []
    for _ in range(_TOPK):
        bv = jnp.max(cur, axis=1, keepdims=True)
        bi = jnp.min(jnp.where(cur == bv, lidx, M), axis=1, keepdims=True)
        top_v.append(bv)
        top_i.append(bi)
        cur = jnp.where(lidx == bi, neg, cur)

    sel = (lidx == top_i[0]) | (lidx == top_i[1]) | (lidx == top_i[2]) | (
        lidx == top_i[3])
    e = jnp.where(sel, jnp.exp(s - top_v[0]), 0.0)
    esum = jnp.sum(e, axis=1, keepdims=True)
    w = [jnp.sum(jnp.where(lidx == top_i[t], e, 0.0), axis=1, keepdims=True)
         / esum for t in range(_TOPK)]

    sidxc = jax.lax.broadcasted_iota(jnp.int32, (M, 1), 0)
    lane = jax.lax.broadcasted_iota(jnp.int32, (8, 128), 1)
    for bb in range(BB):
        wcol = jnp.where(sidxc == top_i[0][bb:bb + 1], w[0][bb:bb + 1], 0.0)
        for t in range(1, _TOPK):
            wcol = wcol + jnp.where(sidxc == top_i[t][bb:bb + 1],
                                    w[t][bb:bb + 1], 0.0)
        out_ref[bb, 0, :] = jnp.sum(x_all[bb] * wcol, axis=0)

        row = jnp.where(
            lane == 0, jnp.broadcast_to(top_i[0][bb:bb + 1], (8, 128)),
            jnp.where(
                lane == 1, jnp.broadcast_to(top_i[1][bb:bb + 1], (8, 128)),
                jnp.where(
                    lane == 2, jnp.broadcast_to(top_i[2][bb:bb + 1], (8, 128)),
                    jnp.where(lane == 3,
                              jnp.broadcast_to(top_i[3][bb:bb + 1], (8, 128)),
                              0))))
        idx_ref[bb] = row


def kernel(hidden_states, attention_mask, query):
    B, S, D = hidden_states.shape
    M = S // _STRIDE

    hs2 = hidden_states.reshape(B * S, D)
    compact = _sc_gather(hs2, B * M, D).reshape(B, M, D)

    mask3 = attention_mask.reshape(B, M, _STRIDE).astype(
        jnp.float32).transpose(0, 2, 1)
    q3 = query.reshape(B, 1, D)

    BB = 4
    retrieved, idx_pad = pl.pallas_call(
        _tc_body,
        grid=(B // BB,),
        in_specs=[
            pl.BlockSpec((BB, M, D), lambda b: (b, 0, 0)),
            pl.BlockSpec((BB, _STRIDE, M), lambda b: (b, 0, 0)),
            pl.BlockSpec((BB, 1, D), lambda b: (b, 0, 0)),
        ],
        out_specs=[
            pl.BlockSpec((BB, 1, D), lambda b: (b, 0, 0)),
            pl.BlockSpec((BB, 8, 128), lambda b: (b, 0, 0)),
        ],
        out_shape=[
            jax.ShapeDtypeStruct((B, 1, D), jnp.float32),
            jax.ShapeDtypeStruct((B, 8, 128), jnp.int32),
        ],
    )(compact, mask3, q3)

    return retrieved.reshape(B, D), idx_pad[:, 0, :_TOPK]


# speedup vs baseline: 13.3024x; 1.0357x over previous
import dataclasses
import functools

import jax
import jax.numpy as jnp
from jax import lax
from jax.experimental import pallas as pl
from jax.experimental.pallas import tpu as pltpu
from jax.experimental.pallas import tpu_sc as plsc

_STRIDE = 32
_TOPK = 4
_L = 16
_NTILE = 32


def _sc_gather(hs2, r0, n_rows, D):
    per_tile = n_rows // _NTILE
    n_chunk = per_tile // _STRIDE

    mesh = plsc.VectorSubcoreMesh(core_axis_name="c", subcore_axis_name="s")
    cp = pltpu.CompilerParams()
    if "needs_layout_passes" in pltpu.CompilerParams.__dataclass_fields__:
        cp = dataclasses.replace(cp, needs_layout_passes=False)

    @functools.partial(
        pl.kernel,
        mesh=mesh,
        compiler_params=cp,
        out_type=jax.ShapeDtypeStruct((n_rows, D), jnp.float32),
        scratch_types=[
            pltpu.VMEM((2 * _L,), jnp.int32),
            pltpu.VMEM((2 * _L,), jnp.int32),
            pltpu.VMEM((_STRIDE, D), jnp.float32),
            pltpu.VMEM((_STRIDE, D), jnp.float32),
            pltpu.SemaphoreType.DMA,
            pltpu.SemaphoreType.DMA,
            pltpu.SemaphoreType.DMA,
            pltpu.SemaphoreType.DMA,
        ],
    )
    def gather_kernel(hs_hbm, out_hbm, idx_a, idx_b, buf_a, buf_b,
                      sg0, sg1, sw0, sw1):
        c = lax.axis_index("c")
        s = lax.axis_index("s")
        w = c * 16 + s
        iota = lax.broadcasted_iota(jnp.int32, (_L,), 0)
        idx = [idx_a, idx_b]
        buf = [buf_a, buf_b]
        sg = [sg0, sg1]
        sw = [sw0, sw1]

        def mkidx(ch, ib):
            f0 = r0 + w * per_tile + ch * _STRIDE
            ib[pl.ds(0, _L)] = (f0 + iota) * _STRIDE
            ib[pl.ds(_L, _L)] = (f0 + _L + iota) * _STRIDE

        gathers = [None] * n_chunk
        writes = [None] * n_chunk
        mkidx(0, idx[0])
        gathers[0] = pltpu.async_copy(hs_hbm.at[idx[0]], buf[0], sg[0])
        for ch in range(n_chunk):
            p = ch % 2
            if ch + 1 < n_chunk:
                mkidx(ch + 1, idx[1 - p])
                if ch >= 1:
                    writes[ch - 1].wait()
                gathers[ch + 1] = pltpu.async_copy(
                    hs_hbm.at[idx[1 - p]], buf[1 - p], sg[1 - p])
            gathers[ch].wait()
            f0 = w * per_tile + ch * _STRIDE
            writes[ch] = pltpu.async_copy(
                buf[p], out_hbm.at[pl.ds(f0, _STRIDE)], sw[p])
        writes[n_chunk - 2].wait()
        writes[n_chunk - 1].wait()

    return gather_kernel(hs2)


def _tc_body(x_ref, m_ref, q_ref, out_ref, idx_ref):
    BB, M, D = x_ref.shape
    x_all = x_ref[...]
    q_all = q_ref[...]
    mrow = m_ref[:, 0, :]

    qn = jnp.sqrt(jnp.sum(q_all * q_all, axis=2, keepdims=True))
    nq = q_all / jnp.maximum(qn, 1e-12)
    n2 = jnp.sum(x_all * x_all, axis=2, keepdims=True)
    nm = x_all / jnp.maximum(jnp.sqrt(n2), 1e-12)
    nm_bf = nm.astype(jnp.bfloat16)
    nq_bf = nq.astype(jnp.bfloat16)
    rows = []
    for bb in range(BB):
        nq8 = jnp.broadcast_to(nq_bf[bb], (8, D))
        s8 = jax.lax.dot_general(
            nq8, nm_bf[bb], (((1,), (1,)), ((), ())),
            preferred_element_type=jnp.float32)
        rows.append(s8[0:1, :])
    neg = jnp.float32(-jnp.inf)
    s = jnp.where(mrow > 0, jnp.concatenate(rows, axis=0), neg)

    lidx = jax.lax.broadcasted_iota(jnp.int32, (BB, M), 1)

    cur = s
    top_i = []
    top_v = []
    for _ in range(_TOPK):
        bv = jnp.max(cur, axis=1, keepdims=True)
        bi = jnp.min(jnp.where(cur == bv, lidx, M), axis=1, keepdims=True)
        top_v.append(bv)
        top_i.append(bi)
        cur = jnp.where(lidx == bi, neg, cur)

    sel = (lidx == top_i[0]) | (lidx == top_i[1]) | (lidx == top_i[2]) | (
        lidx == top_i[3])
    e = jnp.where(sel, jnp.exp(s - top_v[0]), 0.0)
    esum = jnp.sum(e, axis=1, keepdims=True)
    w = [jnp.sum(jnp.where(lidx == top_i[t], e, 0.0), axis=1, keepdims=True)
         / esum for t in range(_TOPK)]

    sidxc = jax.lax.broadcasted_iota(jnp.int32, (M, 1), 0)
    lane = jax.lax.broadcasted_iota(jnp.int32, (8, 128), 1)
    for bb in range(BB):
        wcol = jnp.where(sidxc == top_i[0][bb:bb + 1], w[0][bb:bb + 1], 0.0)
        for t in range(1, _TOPK):
            wcol = wcol + jnp.where(sidxc == top_i[t][bb:bb + 1],
                                    w[t][bb:bb + 1], 0.0)
        out_ref[bb, 0, :] = jnp.sum(x_all[bb] * wcol, axis=0)

        row = jnp.where(
            lane == 0, jnp.broadcast_to(top_i[0][bb:bb + 1], (8, 128)),
            jnp.where(
                lane == 1, jnp.broadcast_to(top_i[1][bb:bb + 1], (8, 128)),
                jnp.where(
                    lane == 2, jnp.broadcast_to(top_i[2][bb:bb + 1], (8, 128)),
                    jnp.where(lane == 3,
                              jnp.broadcast_to(top_i[3][bb:bb + 1], (8, 128)),
                              0))))
        idx_ref[bb] = row


def _tc_call(compact, mask3, q3):
    nb, M, D = compact.shape
    BB = 4
    return pl.pallas_call(
        _tc_body,
        grid=(nb // BB,),
        in_specs=[
            pl.BlockSpec((BB, M, D), lambda b: (b, 0, 0)),
            pl.BlockSpec((BB, _STRIDE, M), lambda b: (b, 0, 0)),
            pl.BlockSpec((BB, 1, D), lambda b: (b, 0, 0)),
        ],
        out_specs=[
            pl.BlockSpec((BB, 1, D), lambda b: (b, 0, 0)),
            pl.BlockSpec((BB, 8, 128), lambda b: (b, 0, 0)),
        ],
        out_shape=[
            jax.ShapeDtypeStruct((nb, 1, D), jnp.float32),
            jax.ShapeDtypeStruct((nb, 8, 128), jnp.int32),
        ],
    )(compact, mask3, q3)


def kernel(hidden_states, attention_mask, query):
    B, S, D = hidden_states.shape
    M = S // _STRIDE

    hs2 = hidden_states.reshape(B * S, D)
    mask3 = attention_mask.reshape(B, M, _STRIDE).astype(
        jnp.float32).transpose(0, 2, 1)
    q3 = query.reshape(B, 1, D)

    G = B // 2
    rets = []
    idxs = []
    for g in range(2):
        compact = _sc_gather(hs2, g * G * M, G * M, D).reshape(G, M, D)
        r, i = _tc_call(compact, mask3[g * G:(g + 1) * G],
                        q3[g * G:(g + 1) * G])
        rets.append(r)
        idxs.append(i)

    retrieved = jnp.concatenate(rets, axis=0).reshape(B, D)
    idx_pad = jnp.concatenate(idxs, axis=0)
    return retrieved, idx_pad[:, 0, :_TOPK]
